# baseline (speedup 1.0000x reference)
import functools
import jax
import jax.numpy as jnp
from jax import lax
from jax.experimental import pallas as pl
from jax.experimental.pallas import tpu as pltpu
from jax.experimental.pallas import tpu_sc as plsc

VOCAB = 1000000
EMB = 64
BATCH = 16384

NC, NS, L = 2, 16, 16
NW = NC * NS
BPW = BATCH // NW

_mesh = plsc.VectorSubcoreMesh(core_axis_name="c", subcore_axis_name="s")


@functools.partial(
    pl.kernel,
    out_type=jax.ShapeDtypeStruct((BATCH,), jnp.float32),
    mesh=_mesh,
    scratch_types=[
        pltpu.VMEM((BPW,), jnp.int32),
        pltpu.VMEM((BPW,), jnp.int32),
        pltpu.VMEM((BPW, EMB), jnp.bfloat16),
        pltpu.VMEM((BPW, EMB), jnp.bfloat16),
        pltpu.VMEM((BPW,), jnp.float32),
        pltpu.VMEM((L * L,), jnp.float32),
        pltpu.SemaphoreType.DMA,
        pltpu.SemaphoreType.DMA,
    ],
    compiler_params=pltpu.CompilerParams(needs_layout_passes=False,
                                         use_tc_tiling_on_sc=False),
)
def _skipgram_kernel(u_hbm, v_hbm, table_hbm, out_hbm,
                     uidx, vidx, urows, vrows, outv, tbuf, sem_u, sem_v):
    wid = lax.axis_index("s") * NC + lax.axis_index("c")
    base = wid * BPW

    pltpu.sync_copy(u_hbm.at[pl.ds(base, BPW)], uidx)
    pltpu.sync_copy(v_hbm.at[pl.ds(base, BPW)], vidx)

    cu = pltpu.async_copy(table_hbm.at[uidx], urows, sem_u)
    cv = pltpu.async_copy(table_hbm.at[vidx], vrows, sem_v)
    cu.wait()
    cv.wait()

    lane = lax.iota(jnp.int32, 16)

    def block(g, _):
        for r in range(L):
            i = g * L + r
            p = jnp.zeros((L,), jnp.float32)
            for k in range(EMB // (2 * L)):
                eu = urows[i, pl.ds(k * 2 * L, 2 * L)]
                ev = vrows[i, pl.ds(k * 2 * L, 2 * L)]
                eu0, eu1 = plsc.unpack(eu, format=plsc.PackFormat.INTERLEAVED)
                ev0, ev1 = plsc.unpack(ev, format=plsc.PackFormat.INTERLEAVED)
                p = p + eu0 * ev0 + eu1 * ev1
            tbuf[pl.ds(r * L, L)] = p
        acc = jnp.zeros((L,), jnp.float32)
        for l in range(L):
            acc = acc + plsc.load_gather(tbuf, [lane * L + l])
        outv[pl.ds(g * L, L)] = acc
        return 0

    lax.fori_loop(0, BPW // L, block, 0)

    pltpu.sync_copy(outv, out_hbm.at[pl.ds(base, BPW)])


def kernel(u, v, emb_weight):
    return _skipgram_kernel(u.astype(jnp.int32), v.astype(jnp.int32),
                            emb_weight.astype(jnp.bfloat16))


# speedup vs baseline: 1.3505x; 1.3505x over previous
import functools
import jax
import jax.numpy as jnp
from jax import lax
from jax.experimental import pallas as pl
from jax.experimental.pallas import tpu as pltpu
from jax.experimental.pallas import tpu_sc as plsc

VOCAB = 1000000
EMB = 64
BATCH = 16384

NC, NS, L = 2, 16, 16
NW = NC * NS
BPW = BATCH // NW

_mesh = plsc.VectorSubcoreMesh(core_axis_name="c", subcore_axis_name="s")


@functools.partial(
    pl.kernel,
    out_type=jax.ShapeDtypeStruct((BATCH,), jnp.float32),
    mesh=_mesh,
    scratch_types=[
        pltpu.VMEM((BPW,), jnp.int32),
        pltpu.VMEM((BPW,), jnp.int32),
        pltpu.VMEM((BPW, EMB), jnp.float32),
        pltpu.VMEM((BPW, EMB), jnp.float32),
        pltpu.VMEM((BPW,), jnp.float32),
        pltpu.VMEM((L * L,), jnp.float32),
        pltpu.SemaphoreType.DMA,
        pltpu.SemaphoreType.DMA,
    ],
    compiler_params=pltpu.CompilerParams(needs_layout_passes=False,
                                         use_tc_tiling_on_sc=False),
)
def _skipgram_kernel(u_hbm, v_hbm, table_hbm, out_hbm,
                     uidx, vidx, urows, vrows, outv, tbuf, sem_u, sem_v):
    wid = lax.axis_index("s") * NC + lax.axis_index("c")
    base = wid * BPW

    pltpu.sync_copy(u_hbm.at[pl.ds(base, BPW)], uidx)
    pltpu.sync_copy(v_hbm.at[pl.ds(base, BPW)], vidx)

    cu = pltpu.async_copy(table_hbm.at[uidx], urows, sem_u)
    cv = pltpu.async_copy(table_hbm.at[vidx], vrows, sem_v)
    cu.wait()
    cv.wait()

    lane = lax.iota(jnp.int32, 16)

    def block(g, _):
        for r in range(L):
            i = g * L + r
            p = jnp.zeros((L,), jnp.float32)
            for k in range(EMB // L):
                eu = urows[i, pl.ds(k * L, L)]
                ev = vrows[i, pl.ds(k * L, L)]
                p = p + eu * ev
            tbuf[pl.ds(r * L, L)] = p
        acc = jnp.zeros((L,), jnp.float32)
        for l in range(L):
            acc = acc + plsc.load_gather(tbuf, [lane * L + l])
        outv[pl.ds(g * L, L)] = acc
        return 0

    lax.fori_loop(0, BPW // L, block, 0)

    pltpu.sync_copy(outv, out_hbm.at[pl.ds(base, BPW)])


def kernel(u, v, emb_weight):
    return _skipgram_kernel(u.astype(jnp.int32), v.astype(jnp.int32),
                            emb_weight)


# speedup vs baseline: 2.8257x; 2.0924x over previous
import functools
import jax
import jax.numpy as jnp
from jax import lax
from jax.experimental import pallas as pl
from jax.experimental.pallas import tpu as pltpu
from jax.experimental.pallas import tpu_sc as plsc

VOCAB = 1000000
EMB = 64
BATCH = 16384

NC, NS, L = 2, 16, 16
NW = NC * NS
BPW = BATCH // NW
CHUNK = 32
NCHUNK = BPW // CHUNK

_mesh = plsc.VectorSubcoreMesh(core_axis_name="c", subcore_axis_name="s")


@functools.partial(
    pl.kernel,
    out_type=jax.ShapeDtypeStruct((BATCH,), jnp.float32),
    mesh=_mesh,
    scratch_types=[
        pltpu.VMEM((BPW,), jnp.int32),
        pltpu.VMEM((BPW,), jnp.int32),
        pltpu.VMEM((CHUNK, 8, EMB), jnp.float32),
        pltpu.VMEM((CHUNK, 8, EMB), jnp.float32),
        pltpu.VMEM((BPW,), jnp.float32),
        pltpu.VMEM((L * L,), jnp.float32),
        pltpu.SemaphoreType.DMA,
        pltpu.SemaphoreType.DMA,
    ],
    compiler_params=pltpu.CompilerParams(needs_layout_passes=False,
                                         use_tc_tiling_on_sc=True),
)
def _skipgram_kernel(u_hbm, v_hbm, tiles_hbm, out_hbm,
                     uidx, vidx, utiles, vtiles, outv, tbuf, sem_u, sem_v):
    wid = lax.axis_index("s") * NC + lax.axis_index("c")
    base = wid * BPW

    pltpu.sync_copy(u_hbm.at[pl.ds(base, BPW)], uidx)
    pltpu.sync_copy(v_hbm.at[pl.ds(base, BPW)], vidx)

    lane = lax.iota(jnp.int32, 16)

    def chunk_body(c, _):
        cbase = c * CHUNK

        def fire(g, _):
            usub = uidx[pl.ds(cbase + g * L, L)]
            vsub = vidx[pl.ds(cbase + g * L, L)]
            ut = lax.shift_right_logical(usub, 3)
            vt = lax.shift_right_logical(vsub, 3)
            for r in range(L):
                i = g * L + r
                pltpu.async_copy(tiles_hbm.at[ut[r]], utiles.at[i], sem_u)
                pltpu.async_copy(tiles_hbm.at[vt[r]], vtiles.at[i], sem_v)
            return 0

        lax.fori_loop(0, CHUNK // L, fire, 0)

        def drain(g, _):
            for r in range(L):
                i = g * L + r
                pltpu.make_async_copy(tiles_hbm.at[0], utiles.at[i],
                                      sem_u).wait()
                pltpu.make_async_copy(tiles_hbm.at[0], vtiles.at[i],
                                      sem_v).wait()
            return 0

        lax.fori_loop(0, CHUNK // L, drain, 0)

        for g in range(CHUNK // L):
            usub = uidx[pl.ds(cbase + g * L, L)] & 7
            vsub = vidx[pl.ds(cbase + g * L, L)] & 7
            for r in range(L):
                i = g * L + r
                su = usub[r]
                sv = vsub[r]
                p = jnp.zeros((L,), jnp.float32)
                for k in range(EMB // L):
                    eu = utiles[i, su, pl.ds(k * L, L)]
                    ev = vtiles[i, sv, pl.ds(k * L, L)]
                    p = p + eu * ev
                tbuf[pl.ds(r * L, L)] = p
            acc = jnp.zeros((L,), jnp.float32)
            for l in range(L):
                acc = acc + plsc.load_gather(tbuf, [lane * L + l])
            outv[pl.ds(cbase + g * L, L)] = acc
        return 0

    lax.fori_loop(0, NCHUNK, chunk_body, 0)

    pltpu.sync_copy(outv, out_hbm.at[pl.ds(base, BPW)])


def kernel(u, v, emb_weight):
    return _skipgram_kernel(u.astype(jnp.int32), v.astype(jnp.int32),
                            emb_weight.reshape(VOCAB // 8, 8, EMB))


# speedup vs baseline: 2.8291x; 1.0012x over previous
import functools
import jax
import jax.numpy as jnp
from jax import lax
from jax.experimental import pallas as pl
from jax.experimental.pallas import tpu as pltpu
from jax.experimental.pallas import tpu_sc as plsc

VOCAB = 1000000
EMB = 64
BATCH = 16384

NC, NS, L = 2, 16, 16
NW = NC * NS
BPW = BATCH // NW
CHUNK = 32
NCHUNK = BPW // CHUNK

_mesh = plsc.VectorSubcoreMesh(core_axis_name="c", subcore_axis_name="s")


@functools.partial(
    pl.kernel,
    out_type=jax.ShapeDtypeStruct((BATCH,), jnp.float32),
    mesh=_mesh,
    scratch_types=[
        pltpu.VMEM((BPW,), jnp.int32),
        pltpu.VMEM((BPW,), jnp.int32),
        pltpu.VMEM((CHUNK, 8, EMB), jnp.float32),
        pltpu.VMEM((CHUNK, 8, EMB), jnp.float32),
        pltpu.VMEM((BPW,), jnp.float32),
        pltpu.VMEM((L * L,), jnp.float32),
        pltpu.SemaphoreType.DMA,
        pltpu.SemaphoreType.DMA,
    ],
    compiler_params=pltpu.CompilerParams(needs_layout_passes=False,
                                         use_tc_tiling_on_sc=True),
)
def _skipgram_kernel(u_hbm, v_hbm, tiles_hbm, out_hbm,
                     uidx, vidx, utiles, vtiles, outv, tbuf, sem_u, sem_v):
    wid = lax.axis_index("s") * NC + lax.axis_index("c")
    base = wid * BPW

    pltpu.sync_copy(u_hbm.at[pl.ds(base, BPW)], uidx)
    pltpu.sync_copy(v_hbm.at[pl.ds(base, BPW)], vidx)

    lane = lax.iota(jnp.int32, 16)

    def chunk_body(c, _):
        cbase = c * CHUNK

        def fire(g, _):
            usub = uidx[pl.ds(cbase + g * L, L)]
            vsub = vidx[pl.ds(cbase + g * L, L)]
            ut = lax.shift_right_logical(usub, 3)
            vt = lax.shift_right_logical(vsub, 3)
            for r in range(L):
                i = g * L + r
                pltpu.async_copy(tiles_hbm.at[ut[r]], utiles.at[i], sem_u)
                pltpu.async_copy(tiles_hbm.at[vt[r]], vtiles.at[i], sem_v)
            return 0

        lax.fori_loop(0, CHUNK // L, fire, 0)

        pltpu.make_async_copy(tiles_hbm.at[pl.ds(0, CHUNK)], utiles,
                              sem_u).wait()
        pltpu.make_async_copy(tiles_hbm.at[pl.ds(0, CHUNK)], vtiles,
                              sem_v).wait()

        for g in range(CHUNK // L):
            usub = uidx[pl.ds(cbase + g * L, L)] & 7
            vsub = vidx[pl.ds(cbase + g * L, L)] & 7
            for r in range(L):
                i = g * L + r
                su = usub[r]
                sv = vsub[r]
                p = jnp.zeros((L,), jnp.float32)
                for k in range(EMB // L):
                    eu = utiles[i, su, pl.ds(k * L, L)]
                    ev = vtiles[i, sv, pl.ds(k * L, L)]
                    p = p + eu * ev
                tbuf[pl.ds(r * L, L)] = p
            acc = jnp.zeros((L,), jnp.float32)
            for l in range(L):
                acc = acc + plsc.load_gather(tbuf, [lane * L + l])
            outv[pl.ds(cbase + g * L, L)] = acc
        return 0

    lax.fori_loop(0, NCHUNK, chunk_body, 0)

    pltpu.sync_copy(outv, out_hbm.at[pl.ds(base, BPW)])


def kernel(u, v, emb_weight):
    return _skipgram_kernel(u.astype(jnp.int32), v.astype(jnp.int32),
                            emb_weight.reshape(VOCAB // 8, 8, EMB))


# speedup vs baseline: 2.9126x; 1.0295x over previous
import functools
import jax
import jax.numpy as jnp
from jax import lax
from jax.experimental import pallas as pl
from jax.experimental.pallas import tpu as pltpu
from jax.experimental.pallas import tpu_sc as plsc

VOCAB = 1000000
EMB = 64
BATCH = 16384

NC, NS, L = 2, 16, 16
NW = NC * NS
BPW = BATCH // NW
CHUNK = 16
NCHUNK = BPW // CHUNK
FP = 8

_mesh = plsc.VectorSubcoreMesh(core_axis_name="c", subcore_axis_name="s")


@functools.partial(
    pl.kernel,
    out_type=jax.ShapeDtypeStruct((BATCH,), jnp.float32),
    mesh=_mesh,
    scratch_types=[
        pltpu.VMEM((BPW + L,), jnp.int32),
        pltpu.VMEM((BPW + L,), jnp.int32),
        pltpu.VMEM((2, CHUNK, 8, EMB), jnp.float32),
        pltpu.VMEM((2, CHUNK, 8, EMB), jnp.float32),
        pltpu.VMEM((BPW,), jnp.float32),
        pltpu.VMEM((L * L,), jnp.float32),
        pltpu.SemaphoreType.DMA,
        pltpu.SemaphoreType.DMA,
        pltpu.SemaphoreType.DMA,
        pltpu.SemaphoreType.DMA,
    ],
    compiler_params=pltpu.CompilerParams(needs_layout_passes=False,
                                         use_tc_tiling_on_sc=True),
)
def _skipgram_kernel(u_hbm, v_hbm, tiles_hbm, out_hbm,
                     uidx, vidx, utiles, vtiles, outv, tbuf,
                     sem_u0, sem_v0, sem_u1, sem_v1):
    wid = lax.axis_index("s") * NC + lax.axis_index("c")
    base = wid * BPW

    pltpu.sync_copy(u_hbm.at[pl.ds(base, BPW)], uidx.at[pl.ds(0, BPW)])
    pltpu.sync_copy(v_hbm.at[pl.ds(base, BPW)], vidx.at[pl.ds(0, BPW)])

    lane = lax.iota(jnp.int32, 16)
    sems = ((sem_u0, sem_v0), (sem_u1, sem_v1))

    def fire(c, buf):
        su, sv = sems[buf]

        def body(g, _):
            off = c * CHUNK + g * FP
            usub = uidx[pl.ds(off, L)]
            vsub = vidx[pl.ds(off, L)]
            ut = lax.shift_right_logical(usub, 3)
            vt = lax.shift_right_logical(vsub, 3)
            for r in range(FP):
                i = g * FP + r
                pltpu.async_copy(tiles_hbm.at[ut[r]], utiles.at[buf, i], su)
                pltpu.async_copy(tiles_hbm.at[vt[r]], vtiles.at[buf, i], sv)
            return 0

        lax.fori_loop(0, CHUNK // FP, body, 0)

    def drain(buf):
        su, sv = sems[buf]
        pltpu.make_async_copy(tiles_hbm.at[pl.ds(0, CHUNK)],
                              utiles.at[buf], su).wait()
        pltpu.make_async_copy(tiles_hbm.at[pl.ds(0, CHUNK)],
                              vtiles.at[buf], sv).wait()

    def compute(c, buf):
        cbase = c * CHUNK
        for g in range(CHUNK // L):
            usub = uidx[pl.ds(cbase + g * L, L)] & 7
            vsub = vidx[pl.ds(cbase + g * L, L)] & 7
            for r in range(L):
                i = g * L + r
                su = usub[r]
                sv = vsub[r]
                p = jnp.zeros((L,), jnp.float32)
                for k in range(EMB // L):
                    eu = utiles[buf, i, su, pl.ds(k * L, L)]
                    ev = vtiles[buf, i, sv, pl.ds(k * L, L)]
                    p = p + eu * ev
                tbuf[pl.ds(r * L, L)] = p
            acc = jnp.zeros((L,), jnp.float32)
            for l in range(L):
                acc = acc + plsc.load_gather(tbuf, [lane * L + l])
            outv[pl.ds(cbase + g * L, L)] = acc

    fire(0, 0)

    def step(h, _):
        c0 = h * 2
        fire(c0 + 1, 1)
        drain(0)
        compute(c0, 0)
        fire(jnp.minimum(c0 + 2, NCHUNK - 2), 0)
        drain(1)
        compute(c0 + 1, 1)
        return 0

    lax.fori_loop(0, NCHUNK // 2, step, 0)
    drain(0)

    pltpu.sync_copy(outv, out_hbm.at[pl.ds(base, BPW)])


def kernel(u, v, emb_weight):
    return _skipgram_kernel(u.astype(jnp.int32), v.astype(jnp.int32),
                            emb_weight.reshape(VOCAB // 8, 8, EMB))
